# baseline (speedup 1.0000x reference)
import functools

import jax
import jax.numpy as jnp
from jax.experimental import pallas as pl
from jax.experimental.pallas import tpu as pltpu

H = 1024
E = 8
TOPK = 2
EPS = 1e-06


def _moe_body(xb_ref, wg_ref, bg_ref, we_ref, be_ref,
              out_ref, aux_ref, wd_ref, w1_ref, w2_ref, i1_ref, i2_ref,
              y_ref):
    e = pl.program_id(0)
    n = xb_ref.shape[0]

    @pl.when(e == 0)
    def _router():
        logits = jax.lax.dot_general(
            xb_ref[...], wg_ref[...], (((1,), (1,)), ((), ())),
            precision=jax.lax.Precision.DEFAULT,
            preferred_element_type=jnp.float32) + bg_ref[...][None, :]
        m = jnp.max(logits, axis=1, keepdims=True)
        ex = jnp.exp(logits - m)
        probs = ex / jnp.sum(ex, axis=1, keepdims=True)
        iota = jax.lax.broadcasted_iota(jnp.int32, (n, E), 1)
        p1 = jnp.max(probs, axis=1, keepdims=True)
        i1 = jnp.min(jnp.where(probs == p1, iota, E), axis=1, keepdims=True)
        masked = jnp.where(iota == i1, -jnp.inf, probs)
        p2 = jnp.max(masked, axis=1, keepdims=True)
        i2 = jnp.min(jnp.where(masked == p2, iota, E), axis=1, keepdims=True)
        denom = p1 + p2 + EPS
        w1_ref[...] = p1 / denom
        w2_ref[...] = p2 / denom
        i1_ref[...] = i1
        i2_ref[...] = i2
        wd_ref[...] = (jnp.where(iota == i1, w1_ref[...], 0.0)
                       + jnp.where(iota == i2, w2_ref[...], 0.0))
        mask = ((iota == i1) | (iota == i2)).astype(jnp.float32)
        usage = jnp.mean(mask, axis=0)
        gates = jnp.mean(probs, axis=0)
        aux_ref[0, 0] = jnp.sum(usage * gates) * E

    @pl.when(e < E)
    def _mm():
        y_ref[e % 2] = jax.lax.dot_general(
            xb_ref[...], we_ref[0].astype(jnp.bfloat16),
            (((1,), (1,)), ((), ())),
            preferred_element_type=jnp.float32)

    @pl.when(e > 0)
    def _combine_prev():
        ep = e - 1
        w_col = (jnp.where(i1_ref[...] == ep, w1_ref[...], 0.0)
                 + jnp.where(i2_ref[...] == ep, w2_ref[...], 0.0))
        contrib = w_col * y_ref[(e - 1) % 2]

        @pl.when(e == 1)
        def _init():
            out_ref[...] = contrib

        @pl.when(e > 1)
        def _acc():
            out_ref[...] += contrib

    @pl.when(e == E)
    def _bias():
        out_ref[...] += jax.lax.dot_general(
            wd_ref[...], be_ref[...], (((1,), (0,)), ((), ())),
            precision=jax.lax.Precision.DEFAULT,
            preferred_element_type=jnp.float32)


@jax.jit
def kernel(x, Wg, bg, We, be):
    b, s, h = x.shape
    xb = x.reshape(-1, h).astype(jnp.bfloat16)
    n = xb.shape[0]

    out, aux = pl.pallas_call(
        _moe_body,
        grid=(E + 1,),
        in_specs=[
            pl.BlockSpec((n, h), lambda e: (0, 0)),
            pl.BlockSpec((E, h), lambda e: (0, 0)),
            pl.BlockSpec((E,), lambda e: (0,)),
            pl.BlockSpec((1, h, h),
                         lambda e: (jnp.minimum(e, E - 1), 0, 0)),
            pl.BlockSpec((E, h), lambda e: (0, 0)),
        ],
        out_specs=[
            pl.BlockSpec((n, h), lambda e: (0, 0)),
            pl.BlockSpec(memory_space=pltpu.SMEM),
        ],
        out_shape=[
            jax.ShapeDtypeStruct((n, h), jnp.float32),
            jax.ShapeDtypeStruct((1, 1), jnp.float32),
        ],
        scratch_shapes=[
            pltpu.VMEM((n, E), jnp.float32),
            pltpu.VMEM((n, 1), jnp.float32),
            pltpu.VMEM((n, 1), jnp.float32),
            pltpu.VMEM((n, 1), jnp.int32),
            pltpu.VMEM((n, 1), jnp.int32),
            pltpu.VMEM((2, n, h), jnp.float32),
        ],
    )(xb, Wg, bg, We, be)

    return out.reshape(b, s, h), aux[0, 0]
